# Initial kernel scaffold; baseline (speedup 1.0000x reference)
import jax
import jax.numpy as jnp
from jax.experimental import pallas as pl


def kernel(x, edge_index, params):
    raise NotImplementedError("write your pallas kernel here")


# speedup vs baseline: 41.9690x; 41.9690x over previous
import functools

import jax
import jax.numpy as jnp
from jax import lax
from jax.experimental import pallas as pl
from jax.experimental.pallas import tpu as pltpu
from jax.experimental.pallas import tpu_sc as plsc

N = 10000
F = 128
H = 8
NL = 3
E2 = 330000
LSTM_H = 192

N2 = 10240
EP = 331776
NC, NS = 2, 16
NW = NC * NS
PER_W = EP // NW
CH = 128
NCHUNK = PER_W // CH
NPS = N2 // NS

BN = 1024
NB = N2 // BN
BE = 5184
NEB = EP // BE

_f32 = jnp.float32


@functools.cache
def _sc_kernels():
    mesh = plsc.VectorSubcoreMesh(
        core_axis_name="c", subcore_axis_name="s",
        num_cores=NC, num_subcores=NS)

    @functools.partial(
        pl.kernel,
        out_type=(jax.ShapeDtypeStruct((EP, F), _f32),
                  jax.ShapeDtypeStruct((EP, F), _f32)),
        mesh=mesh,
        scratch_types=[
            pltpu.VMEM((PER_W,), jnp.int32),
            pltpu.VMEM((PER_W,), jnp.int32),
            pltpu.VMEM((2, CH, F), _f32),
            pltpu.VMEM((2, CH, F), _f32),
            pltpu.SemaphoreType.DMA,
            pltpu.SemaphoreType.DMA,
            pltpu.SemaphoreType.DMA,
            pltpu.SemaphoreType.DMA,
        ],
    )
    def edge_gather(xl_hbm, xr_hbm, src_hbm, dst_hbm, gl_hbm, gr_hbm,
                    si, di, bl, br, sg0, sg1, sw0, sw1):
        wid = lax.axis_index("s") * NC + lax.axis_index("c")
        base = wid * PER_W

        pltpu.sync_copy(src_hbm.at[pl.ds(base, PER_W)], si)
        pltpu.sync_copy(dst_hbm.at[pl.ds(base, PER_W)], di)

        def fire_gather(c, b, sem):
            pltpu.async_copy(xl_hbm.at[si.at[pl.ds(c * CH, CH)]], bl.at[b], sem)
            pltpu.async_copy(xr_hbm.at[di.at[pl.ds(c * CH, CH)]], br.at[b], sem)

        def wait_gather(c, b, sem):
            pltpu.make_async_copy(
                xl_hbm.at[si.at[pl.ds(c * CH, CH)]], bl.at[b], sem).wait()
            pltpu.make_async_copy(
                xr_hbm.at[di.at[pl.ds(c * CH, CH)]], br.at[b], sem).wait()

        def fire_wb(c, b, sem):
            pltpu.async_copy(bl.at[b], gl_hbm.at[pl.ds(base + c * CH, CH)], sem)
            pltpu.async_copy(br.at[b], gr_hbm.at[pl.ds(base + c * CH, CH)], sem)

        def wait_wb(c, b, sem):
            pltpu.make_async_copy(
                bl.at[b], gl_hbm.at[pl.ds(base + c * CH, CH)], sem).wait()
            pltpu.make_async_copy(
                br.at[b], gr_hbm.at[pl.ds(base + c * CH, CH)], sem).wait()

        fire_gather(0, 0, sg0)

        def body(pi, carry):
            c = 2 * pi
            fire_gather(c + 1, 1, sg1)
            wait_gather(c, 0, sg0)
            fire_wb(c, 0, sw0)
            wait_gather(c + 1, 1, sg1)
            fire_wb(c + 1, 1, sw1)
            wait_wb(c, 0, sw0)
            fire_gather(c + 2, 0, sg0)
            wait_wb(c + 1, 1, sw1)
            return carry

        lax.fori_loop(0, (NCHUNK - 1) // 2, body, 0)
        wait_gather(NCHUNK - 1, 0, sg0)
        fire_wb(NCHUNK - 1, 0, sw0)
        wait_wb(NCHUNK - 1, 0, sw0)

    def make_scatter(width):
        @functools.partial(
            pl.kernel,
            out_type=jax.ShapeDtypeStruct((NC * N2, width), _f32),
            mesh=mesh,
            scratch_types=[
                pltpu.VMEM((2, CH), jnp.int32),
                pltpu.VMEM((2, CH, width), _f32),
                pltpu.VMEM_SHARED((N2, width), _f32),
                pltpu.SemaphoreType.DMA,
                pltpu.SemaphoreType.DMA,
            ],
        )
        def edge_scatter(w_hbm, dst_hbm, zf_hbm, out_hbm, di, wb, acc,
                         sl0, sl1):
            cid = lax.axis_index("c")
            sid = lax.axis_index("s")
            wid = sid * NC + cid
            rows0 = sid * NPS
            base = wid * PER_W

            def fire_load(c, b, sem):
                pltpu.async_copy(w_hbm.at[pl.ds(base + c * CH, CH)],
                                 wb.at[b], sem)
                pltpu.async_copy(dst_hbm.at[pl.ds(base + c * CH, CH)],
                                 di.at[b], sem)

            def wait_load(c, b, sem):
                pltpu.make_async_copy(
                    w_hbm.at[pl.ds(base + c * CH, CH)], wb.at[b], sem).wait()
                pltpu.make_async_copy(
                    dst_hbm.at[pl.ds(base + c * CH, CH)], di.at[b], sem).wait()

            fire_load(0, 0, sl0)
            pltpu.sync_copy(zf_hbm.at[pl.ds(rows0, NPS)],
                            acc.at[pl.ds(rows0, NPS)])
            plsc.subcore_barrier()

            def body(pi, carry):
                c = 2 * pi
                wait_load(c, 0, sl0)
                fire_load(c + 1, 1, sl1)
                pltpu.sync_copy(wb.at[0], acc.at[di.at[0]], add=True)
                fire_load(c + 2, 0, sl0)
                wait_load(c + 1, 1, sl1)
                pltpu.sync_copy(wb.at[1], acc.at[di.at[1]], add=True)
                return carry

            lax.fori_loop(0, (NCHUNK - 1) // 2, body, 0)
            wait_load(NCHUNK - 1, 0, sl0)
            pltpu.sync_copy(wb.at[0], acc.at[di.at[0]], add=True)
            plsc.subcore_barrier()

            pltpu.sync_copy(acc.at[pl.ds(rows0, NPS)],
                            out_hbm.at[pl.ds(cid * N2 + rows0, NPS)])

        return edge_scatter

    return edge_gather, make_scatter(F), make_scatter(16)


def _edge_gather(xl, xr, src, dst):
    return _sc_kernels()[0](xl, xr, src, dst)


def _edge_scatter_f(w, dst, zf):
    return _sc_kernels()[1](w, dst, zf)


def _edge_scatter_d(pp, dst, zd):
    return _sc_kernels()[2](pp, dst, zd)


def _leaky(v, s):
    return jnp.where(v > 0, v, s * v)


def _mm2_body(h_ref, wl_ref, bl_ref, wr_ref, br_ref, xl_ref, xr_ref):
    h = h_ref[...]
    xl_ref[...] = jnp.dot(h, wl_ref[...], preferred_element_type=_f32) + bl_ref[...]
    xr_ref[...] = jnp.dot(h, wr_ref[...], preferred_element_type=_f32) + br_ref[...]


def _mm2(h, Wl, bl, Wr, br):
    return pl.pallas_call(
        _mm2_body,
        grid=(NB,),
        in_specs=[pl.BlockSpec((BN, F), lambda i: (i, 0)),
                  pl.BlockSpec((F, F), lambda i: (0, 0)),
                  pl.BlockSpec((1, F), lambda i: (0, 0)),
                  pl.BlockSpec((F, F), lambda i: (0, 0)),
                  pl.BlockSpec((1, F), lambda i: (0, 0))],
        out_specs=[pl.BlockSpec((BN, F), lambda i: (i, 0)),
                   pl.BlockSpec((BN, F), lambda i: (i, 0))],
        out_shape=[jax.ShapeDtypeStruct((N2, F), _f32)] * 2,
    )(h, Wl, bl, Wr, br)


def _alpha_body(gl_ref, gr_ref, a16_ref, al_ref, gmax_ref):
    s = gl_ref[...] + gr_ref[...]
    e = _leaky(s, 0.2)
    al = jnp.dot(e, a16_ref[...], preferred_element_type=_f32,
                 precision=lax.Precision.HIGHEST)
    al_ref[...] = al

    @pl.when(pl.program_id(0) == 0)
    def _():
        gmax_ref[...] = jnp.full((1, 1), -1e30, _f32)

    gmax_ref[...] = jnp.maximum(gmax_ref[...], jnp.max(al))


def _alpha_call(gl, gr, A16):
    return pl.pallas_call(
        _alpha_body,
        grid=(NEB,),
        in_specs=[pl.BlockSpec((BE, F), lambda i: (i, 0)),
                  pl.BlockSpec((BE, F), lambda i: (i, 0)),
                  pl.BlockSpec((F, 16), lambda i: (0, 0))],
        out_specs=[pl.BlockSpec((BE, 16), lambda i: (i, 0)),
                   pl.BlockSpec((1, 1), lambda i: (0, 0))],
        out_shape=[jax.ShapeDtypeStruct((EP, 16), _f32),
                   jax.ShapeDtypeStruct((1, 1), _f32)],
    )(gl, gr, A16)


def _wp_body(gl_ref, al_ref, gmax_ref, b16_ref, w_ref, pp_ref):
    p = jnp.exp(al_ref[...] - gmax_ref[0, 0])
    pp_ref[...] = p
    pb = jnp.dot(p, b16_ref[...], preferred_element_type=_f32,
                 precision=lax.Precision.HIGHEST)
    w_ref[...] = gl_ref[...] * pb


def _wp_call(gl, al, gmax, B16):
    return pl.pallas_call(
        _wp_body,
        grid=(NEB,),
        in_specs=[pl.BlockSpec((BE, F), lambda i: (i, 0)),
                  pl.BlockSpec((BE, 16), lambda i: (i, 0)),
                  pl.BlockSpec((1, 1), lambda i: (0, 0)),
                  pl.BlockSpec((16, F), lambda i: (0, 0))],
        out_specs=[pl.BlockSpec((BE, F), lambda i: (i, 0)),
                   pl.BlockSpec((BE, 16), lambda i: (i, 0))],
        out_shape=[jax.ShapeDtypeStruct((EP, F), _f32),
                   jax.ShapeDtypeStruct((EP, 16), _f32)],
    )(gl, al, gmax, B16)


def _fin_body(a0_ref, a1_ref, d0_ref, d1_ref, b16_ref, bias_ref, h_ref):
    den = jnp.dot(d0_ref[...] + d1_ref[...], b16_ref[...],
                  preferred_element_type=_f32,
                  precision=lax.Precision.HIGHEST)
    o = (a0_ref[...] + a1_ref[...]) / (den + 1e-16) + bias_ref[...]
    h_ref[...] = _leaky(o, 0.01)


def _fin_call(accs, accds, B16, bias):
    return pl.pallas_call(
        _fin_body,
        grid=(NB,),
        in_specs=[pl.BlockSpec((BN, F), lambda i: (i, 0)),
                  pl.BlockSpec((BN, F), lambda i: (i + NB, 0)),
                  pl.BlockSpec((BN, 16), lambda i: (i, 0)),
                  pl.BlockSpec((BN, 16), lambda i: (i + NB, 0)),
                  pl.BlockSpec((16, F), lambda i: (0, 0)),
                  pl.BlockSpec((1, F), lambda i: (0, 0))],
        out_specs=[pl.BlockSpec((BN, F), lambda i: (i, 0))],
        out_shape=[jax.ShapeDtypeStruct((N2, F), _f32)],
    )(accs, accs, accds, accds, B16, bias)[0]


def _sig(v):
    return 1.0 / (1.0 + jnp.exp(-v))


def _tail_body(x1_ref, x2_ref, x3_ref, wif_ref, whf_ref, bf_ref,
               wib_ref, whb_ref, bb_ref, jkw_ref, jkb_ref,
               lin_ref, linb_ref, fc1_ref, fc1b_ref, fc2_ref, fc2b_ref,
               out_ref):
    xs = [x1_ref[...], x2_ref[...], x3_ref[...]]

    def run_lstm(wi_ref, wh_ref, b_ref, order):
        wi = wi_ref[...]
        wh = wh_ref[...]
        b = b_ref[...]
        h = jnp.zeros((BN, LSTM_H), _f32)
        c = jnp.zeros((BN, LSTM_H), _f32)
        outs = [None] * NL
        for t in order:
            g = (jnp.dot(xs[t], wi, preferred_element_type=_f32)
                 + jnp.dot(h, wh, preferred_element_type=_f32) + b)
            ig = _sig(g[:, 0:LSTM_H])
            fg = _sig(g[:, LSTM_H:2 * LSTM_H])
            gg = jnp.tanh(g[:, 2 * LSTM_H:3 * LSTM_H])
            og = _sig(g[:, 3 * LSTM_H:4 * LSTM_H])
            c = fg * c + ig * gg
            h = og * jnp.tanh(c)
            outs[t] = h
        return outs

    hf = run_lstm(wif_ref, whf_ref, bf_ref, range(NL))
    hb = run_lstm(wib_ref, whb_ref, bb_ref, range(NL - 1, -1, -1))

    jkw = jkw_ref[...]
    jkb = jkb_ref[0, 0]
    a = [jnp.sum(jnp.concatenate([hf[t], hb[t]], axis=1) * jkw,
                 axis=1, keepdims=True) + jkb for t in range(NL)]
    amax = jnp.maximum(jnp.maximum(a[0], a[1]), a[2])
    ex = [jnp.exp(at - amax) for at in a]
    tot = ex[0] + ex[1] + ex[2]
    jk = (ex[0] * xs[0] + ex[1] * xs[1] + ex[2] * xs[2]) / tot

    outm = jnp.dot(jk, lin_ref[...], preferred_element_type=_f32) + linb_ref[...]
    v = jnp.sum(outm * fc1_ref[...], axis=1, keepdims=True) + fc1b_ref[0, 0]
    v = _leaky(v, 0.01)
    part = jnp.sum(v[:, 0] * fc2_ref[0, :])

    @pl.when(pl.program_id(0) == 0)
    def _():
        out_ref[...] = fc2b_ref[...]

    out_ref[...] += part


def _tail_call(xs1, xs2, xs3, wif, whf, bf, wib, whb, bb,
               jkw, jkb, lin, linb, fc1, fc1b, fc2, fc2b):
    blk = lambda r, c: pl.BlockSpec((r, c), lambda i: (0, 0))
    return pl.pallas_call(
        _tail_body,
        grid=(NB,),
        in_specs=[pl.BlockSpec((BN, F), lambda i: (i, 0)),
                  pl.BlockSpec((BN, F), lambda i: (i, 0)),
                  pl.BlockSpec((BN, F), lambda i: (i, 0)),
                  blk(F, 4 * LSTM_H), blk(LSTM_H, 4 * LSTM_H), blk(1, 4 * LSTM_H),
                  blk(F, 4 * LSTM_H), blk(LSTM_H, 4 * LSTM_H), blk(1, 4 * LSTM_H),
                  blk(1, 2 * LSTM_H), blk(1, 1),
                  blk(F, F), blk(1, F), blk(1, F), blk(1, 1),
                  pl.BlockSpec((1, BN), lambda i: (0, i)), blk(1, 1)],
        out_specs=[pl.BlockSpec((1, 1), lambda i: (0, 0))],
        out_shape=[jax.ShapeDtypeStruct((1, 1), _f32)],
    )(xs1, xs2, xs3, wif, whf, bf, wib, whb, bb,
      jkw, jkb, lin, linb, fc1, fc1b, fc2, fc2b)[0]


def kernel(x, edge_index, params):
    p = params
    loop = jnp.arange(N, dtype=jnp.int32)
    pad = jnp.full((EP - E2,), N, jnp.int32)
    src = jnp.concatenate([edge_index[0].astype(jnp.int32), loop, pad])
    dst = jnp.concatenate([edge_index[1].astype(jnp.int32), loop, pad])

    x_pad = jnp.zeros((N2, F), _f32).at[:N].set(x)
    zf = jnp.zeros((N2, F), _f32)
    zd = jnp.zeros((N2, 16), _f32)

    kron_a = jnp.kron(jnp.eye(H, dtype=_f32), jnp.ones((F // H, 1), _f32))
    kron_b = jnp.kron(jnp.eye(H, dtype=_f32), jnp.ones((1, F // H), _f32))
    B16 = jnp.zeros((16, F), _f32).at[:H].set(kron_b)

    h = x_pad
    xs = []
    for l in range(NL):
        A16 = jnp.zeros((F, 16), _f32).at[:, :H].set(
            kron_a * p['att%d' % l].reshape(F, 1))
        xl, xr = _mm2(h, p['Wl%d' % l], p['bl%d' % l].reshape(1, F),
                      p['Wr%d' % l], p['br%d' % l].reshape(1, F))
        gl, gr = _edge_gather(xl, xr, src, dst)
        al, gmax = _alpha_call(gl, gr, A16)
        w, pp = _wp_call(gl, al, gmax, B16)
        accs = _edge_scatter_f(w, dst, zf)
        accds = _edge_scatter_d(pp, dst, zd)
        h = _fin_call(accs, accds, B16, p['bias%d' % l].reshape(1, F))
        xs.append(h)

    bf = (p['bih_fwd'] + p['bhh_fwd']).reshape(1, 4 * LSTM_H)
    bb = (p['bih_bwd'] + p['bhh_bwd']).reshape(1, 4 * LSTM_H)
    fc2_pad = jnp.zeros((1, N2), _f32).at[:, :N].set(p['fc2_w'])

    out = _tail_call(
        xs[0], xs[1], xs[2],
        p['Wih_fwd'].T, p['Whh_fwd'].T, bf,
        p['Wih_bwd'].T, p['Whh_bwd'].T, bb,
        p['jk_att_w'], p['jk_att_b'].reshape(1, 1),
        p['lin_w'].T, p['lin_b'].reshape(1, F),
        p['fc1_w'], p['fc1_b'].reshape(1, 1),
        fc2_pad, p['fc2_b'].reshape(1, 1))
    return out.reshape(1)


# speedup vs baseline: 42.9790x; 1.0241x over previous
import functools

import jax
import jax.numpy as jnp
from jax import lax
from jax.experimental import pallas as pl
from jax.experimental.pallas import tpu as pltpu
from jax.experimental.pallas import tpu_sc as plsc

N = 10000
F = 128
H = 8
NL = 3
E2 = 330000
LSTM_H = 192

N2 = 10240
EP = 331776
NC, NS = 2, 16
NW = NC * NS
PER_W = EP // NW
CH = 128
NCHUNK = PER_W // CH
NPS = N2 // NS

BN = 1024
NB = N2 // BN
BE = 5184
NEB = EP // BE

_f32 = jnp.float32


@functools.cache
def _sc_kernels():
    mesh = plsc.VectorSubcoreMesh(
        core_axis_name="c", subcore_axis_name="s",
        num_cores=NC, num_subcores=NS)

    @functools.partial(
        pl.kernel,
        out_type=(jax.ShapeDtypeStruct((EP, F), _f32),
                  jax.ShapeDtypeStruct((EP, F), _f32)),
        mesh=mesh,
        scratch_types=[
            pltpu.VMEM((PER_W,), jnp.int32),
            pltpu.VMEM((PER_W,), jnp.int32),
            pltpu.VMEM((3, CH, F), _f32),
            pltpu.VMEM((3, CH, F), _f32),
            pltpu.SemaphoreType.DMA,
            pltpu.SemaphoreType.DMA,
            pltpu.SemaphoreType.DMA,
            pltpu.SemaphoreType.DMA,
            pltpu.SemaphoreType.DMA,
            pltpu.SemaphoreType.DMA,
        ],
    )
    def edge_gather(xl_hbm, xr_hbm, src_hbm, dst_hbm, gl_hbm, gr_hbm,
                    si, di, bl, br, sg0, sg1, sg2, sw0, sw1, sw2):
        wid = lax.axis_index("s") * NC + lax.axis_index("c")
        base = wid * PER_W

        pltpu.sync_copy(src_hbm.at[pl.ds(base, PER_W)], si)
        pltpu.sync_copy(dst_hbm.at[pl.ds(base, PER_W)], di)

        def fire_gather(c, b, sem):
            pltpu.async_copy(xl_hbm.at[si.at[pl.ds(c * CH, CH)]], bl.at[b], sem)
            pltpu.async_copy(xr_hbm.at[di.at[pl.ds(c * CH, CH)]], br.at[b], sem)

        def wait_gather(c, b, sem):
            pltpu.make_async_copy(
                xl_hbm.at[si.at[pl.ds(c * CH, CH)]], bl.at[b], sem).wait()
            pltpu.make_async_copy(
                xr_hbm.at[di.at[pl.ds(c * CH, CH)]], br.at[b], sem).wait()

        def fire_wb(c, b, sem):
            pltpu.async_copy(bl.at[b], gl_hbm.at[pl.ds(base + c * CH, CH)], sem)
            pltpu.async_copy(br.at[b], gr_hbm.at[pl.ds(base + c * CH, CH)], sem)

        def wait_wb(c, b, sem):
            pltpu.make_async_copy(
                bl.at[b], gl_hbm.at[pl.ds(base + c * CH, CH)], sem).wait()
            pltpu.make_async_copy(
                br.at[b], gr_hbm.at[pl.ds(base + c * CH, CH)], sem).wait()

        sg = (sg0, sg1, sg2)
        sw = (sw0, sw1, sw2)

        fire_gather(0, 0, sg0)
        fire_gather(1, 1, sg1)

        wait_gather(0, 0, sg0)
        fire_wb(0, 0, sw0)
        fire_gather(2, 2, sg2)
        wait_gather(1, 1, sg1)
        fire_wb(1, 1, sw1)
        wait_wb(0, 0, sw0)
        fire_gather(3, 0, sg0)
        wait_gather(2, 2, sg2)
        fire_wb(2, 2, sw2)
        wait_wb(1, 1, sw1)
        fire_gather(4, 1, sg1)

        def body(pi, carry):
            c0 = 3 * pi
            for k in range(3):
                c = c0 + k
                bcur = k
                bnxt = (k + 2) % 3
                wait_gather(c, bcur, sg[bcur])
                fire_wb(c, bcur, sw[bcur])
                wait_wb(c - 1, bnxt, sw[bnxt])
                fire_gather(c + 2, bnxt, sg[bnxt])
            return carry

        lax.fori_loop(1, (NCHUNK - 3) // 3, body, 0)

        c = NCHUNK - 3
        wait_gather(c, 0, sg0)
        fire_wb(c, 0, sw0)
        wait_wb(c - 1, 2, sw2)
        fire_gather(c + 2, 2, sg2)
        wait_gather(c + 1, 1, sg1)
        fire_wb(c + 1, 1, sw1)
        wait_gather(c + 2, 2, sg2)
        fire_wb(c + 2, 2, sw2)
        wait_wb(c, 0, sw0)
        wait_wb(c + 1, 1, sw1)
        wait_wb(c + 2, 2, sw2)

    def make_scatter(width):
        @functools.partial(
            pl.kernel,
            out_type=jax.ShapeDtypeStruct((NC * N2, width), _f32),
            mesh=mesh,
            scratch_types=[
                pltpu.VMEM((2, CH), jnp.int32),
                pltpu.VMEM((2, CH, width), _f32),
                pltpu.VMEM_SHARED((N2, width), _f32),
                pltpu.SemaphoreType.DMA,
                pltpu.SemaphoreType.DMA,
            ],
        )
        def edge_scatter(w_hbm, dst_hbm, zf_hbm, out_hbm, di, wb, acc,
                         sl0, sl1):
            cid = lax.axis_index("c")
            sid = lax.axis_index("s")
            wid = sid * NC + cid
            rows0 = sid * NPS
            base = wid * PER_W

            def fire_load(c, b, sem):
                pltpu.async_copy(w_hbm.at[pl.ds(base + c * CH, CH)],
                                 wb.at[b], sem)
                pltpu.async_copy(dst_hbm.at[pl.ds(base + c * CH, CH)],
                                 di.at[b], sem)

            def wait_load(c, b, sem):
                pltpu.make_async_copy(
                    w_hbm.at[pl.ds(base + c * CH, CH)], wb.at[b], sem).wait()
                pltpu.make_async_copy(
                    dst_hbm.at[pl.ds(base + c * CH, CH)], di.at[b], sem).wait()

            fire_load(0, 0, sl0)
            pltpu.sync_copy(zf_hbm.at[pl.ds(rows0, NPS)],
                            acc.at[pl.ds(rows0, NPS)])
            plsc.subcore_barrier()

            def body(pi, carry):
                c = 2 * pi
                wait_load(c, 0, sl0)
                fire_load(c + 1, 1, sl1)
                pltpu.sync_copy(wb.at[0], acc.at[di.at[0]], add=True)
                fire_load(c + 2, 0, sl0)
                wait_load(c + 1, 1, sl1)
                pltpu.sync_copy(wb.at[1], acc.at[di.at[1]], add=True)
                return carry

            lax.fori_loop(0, (NCHUNK - 1) // 2, body, 0)
            wait_load(NCHUNK - 1, 0, sl0)
            pltpu.sync_copy(wb.at[0], acc.at[di.at[0]], add=True)
            plsc.subcore_barrier()

            pltpu.sync_copy(acc.at[pl.ds(rows0, NPS)],
                            out_hbm.at[pl.ds(cid * N2 + rows0, NPS)])

        return edge_scatter

    return edge_gather, make_scatter(F), make_scatter(16)


def _edge_gather(xl, xr, src, dst):
    return _sc_kernels()[0](xl, xr, src, dst)


def _edge_scatter_f(w, dst, zf):
    return _sc_kernels()[1](w, dst, zf)


def _edge_scatter_d(pp, dst, zd):
    return _sc_kernels()[2](pp, dst, zd)


def _leaky(v, s):
    return jnp.where(v > 0, v, s * v)


def _mm2_body(h_ref, wl_ref, bl_ref, wr_ref, br_ref, xl_ref, xr_ref):
    h = h_ref[...]
    xl_ref[...] = jnp.dot(h, wl_ref[...], preferred_element_type=_f32) + bl_ref[...]
    xr_ref[...] = jnp.dot(h, wr_ref[...], preferred_element_type=_f32) + br_ref[...]


def _mm2(h, Wl, bl, Wr, br):
    return pl.pallas_call(
        _mm2_body,
        grid=(NB,),
        in_specs=[pl.BlockSpec((BN, F), lambda i: (i, 0)),
                  pl.BlockSpec((F, F), lambda i: (0, 0)),
                  pl.BlockSpec((1, F), lambda i: (0, 0)),
                  pl.BlockSpec((F, F), lambda i: (0, 0)),
                  pl.BlockSpec((1, F), lambda i: (0, 0))],
        out_specs=[pl.BlockSpec((BN, F), lambda i: (i, 0)),
                   pl.BlockSpec((BN, F), lambda i: (i, 0))],
        out_shape=[jax.ShapeDtypeStruct((N2, F), _f32)] * 2,
    )(h, Wl, bl, Wr, br)


def _alpha_body(gl_ref, gr_ref, a16_ref, al_ref, gmax_ref):
    s = gl_ref[...] + gr_ref[...]
    e = _leaky(s, 0.2)
    al = jnp.dot(e, a16_ref[...], preferred_element_type=_f32,
                 precision=lax.Precision.HIGHEST)
    al_ref[...] = al

    @pl.when(pl.program_id(0) == 0)
    def _():
        gmax_ref[...] = jnp.full((1, 1), -1e30, _f32)

    gmax_ref[...] = jnp.maximum(gmax_ref[...], jnp.max(al))


def _alpha_call(gl, gr, A16):
    return pl.pallas_call(
        _alpha_body,
        grid=(NEB,),
        in_specs=[pl.BlockSpec((BE, F), lambda i: (i, 0)),
                  pl.BlockSpec((BE, F), lambda i: (i, 0)),
                  pl.BlockSpec((F, 16), lambda i: (0, 0))],
        out_specs=[pl.BlockSpec((BE, 16), lambda i: (i, 0)),
                   pl.BlockSpec((1, 1), lambda i: (0, 0))],
        out_shape=[jax.ShapeDtypeStruct((EP, 16), _f32),
                   jax.ShapeDtypeStruct((1, 1), _f32)],
    )(gl, gr, A16)


def _wp_body(gl_ref, al_ref, gmax_ref, b16_ref, w_ref, pp_ref):
    p = jnp.exp(al_ref[...] - gmax_ref[0, 0])
    pp_ref[...] = p
    pb = jnp.dot(p, b16_ref[...], preferred_element_type=_f32,
                 precision=lax.Precision.HIGHEST)
    w_ref[...] = gl_ref[...] * pb


def _wp_call(gl, al, gmax, B16):
    return pl.pallas_call(
        _wp_body,
        grid=(NEB,),
        in_specs=[pl.BlockSpec((BE, F), lambda i: (i, 0)),
                  pl.BlockSpec((BE, 16), lambda i: (i, 0)),
                  pl.BlockSpec((1, 1), lambda i: (0, 0)),
                  pl.BlockSpec((16, F), lambda i: (0, 0))],
        out_specs=[pl.BlockSpec((BE, F), lambda i: (i, 0)),
                   pl.BlockSpec((BE, 16), lambda i: (i, 0))],
        out_shape=[jax.ShapeDtypeStruct((EP, F), _f32),
                   jax.ShapeDtypeStruct((EP, 16), _f32)],
    )(gl, al, gmax, B16)


def _fin_body(a0_ref, a1_ref, d0_ref, d1_ref, b16_ref, bias_ref, h_ref):
    den = jnp.dot(d0_ref[...] + d1_ref[...], b16_ref[...],
                  preferred_element_type=_f32,
                  precision=lax.Precision.HIGHEST)
    o = (a0_ref[...] + a1_ref[...]) / (den + 1e-16) + bias_ref[...]
    h_ref[...] = _leaky(o, 0.01)


def _fin_call(accs, accds, B16, bias):
    return pl.pallas_call(
        _fin_body,
        grid=(NB,),
        in_specs=[pl.BlockSpec((BN, F), lambda i: (i, 0)),
                  pl.BlockSpec((BN, F), lambda i: (i + NB, 0)),
                  pl.BlockSpec((BN, 16), lambda i: (i, 0)),
                  pl.BlockSpec((BN, 16), lambda i: (i + NB, 0)),
                  pl.BlockSpec((16, F), lambda i: (0, 0)),
                  pl.BlockSpec((1, F), lambda i: (0, 0))],
        out_specs=[pl.BlockSpec((BN, F), lambda i: (i, 0))],
        out_shape=[jax.ShapeDtypeStruct((N2, F), _f32)],
    )(accs, accs, accds, accds, B16, bias)[0]


def _sig(v):
    return 1.0 / (1.0 + jnp.exp(-v))


def _tail_body(x1_ref, x2_ref, x3_ref, wif_ref, whf_ref, bf_ref,
               wib_ref, whb_ref, bb_ref, jkw_ref, jkb_ref,
               lin_ref, linb_ref, fc1_ref, fc1b_ref, fc2_ref, fc2b_ref,
               out_ref):
    xs = [x1_ref[...], x2_ref[...], x3_ref[...]]

    def run_lstm(wi_ref, wh_ref, b_ref, order):
        wi = wi_ref[...]
        wh = wh_ref[...]
        b = b_ref[...]
        h = jnp.zeros((BN, LSTM_H), _f32)
        c = jnp.zeros((BN, LSTM_H), _f32)
        outs = [None] * NL
        for t in order:
            g = (jnp.dot(xs[t], wi, preferred_element_type=_f32)
                 + jnp.dot(h, wh, preferred_element_type=_f32) + b)
            ig = _sig(g[:, 0:LSTM_H])
            fg = _sig(g[:, LSTM_H:2 * LSTM_H])
            gg = jnp.tanh(g[:, 2 * LSTM_H:3 * LSTM_H])
            og = _sig(g[:, 3 * LSTM_H:4 * LSTM_H])
            c = fg * c + ig * gg
            h = og * jnp.tanh(c)
            outs[t] = h
        return outs

    hf = run_lstm(wif_ref, whf_ref, bf_ref, range(NL))
    hb = run_lstm(wib_ref, whb_ref, bb_ref, range(NL - 1, -1, -1))

    jkw = jkw_ref[...]
    jkb = jkb_ref[0, 0]
    a = [jnp.sum(jnp.concatenate([hf[t], hb[t]], axis=1) * jkw,
                 axis=1, keepdims=True) + jkb for t in range(NL)]
    amax = jnp.maximum(jnp.maximum(a[0], a[1]), a[2])
    ex = [jnp.exp(at - amax) for at in a]
    tot = ex[0] + ex[1] + ex[2]
    jk = (ex[0] * xs[0] + ex[1] * xs[1] + ex[2] * xs[2]) / tot

    outm = jnp.dot(jk, lin_ref[...], preferred_element_type=_f32) + linb_ref[...]
    v = jnp.sum(outm * fc1_ref[...], axis=1, keepdims=True) + fc1b_ref[0, 0]
    v = _leaky(v, 0.01)
    part = jnp.sum(v[:, 0] * fc2_ref[0, :])

    @pl.when(pl.program_id(0) == 0)
    def _():
        out_ref[...] = fc2b_ref[...]

    out_ref[...] += part


def _tail_call(xs1, xs2, xs3, wif, whf, bf, wib, whb, bb,
               jkw, jkb, lin, linb, fc1, fc1b, fc2, fc2b):
    blk = lambda r, c: pl.BlockSpec((r, c), lambda i: (0, 0))
    return pl.pallas_call(
        _tail_body,
        grid=(NB,),
        in_specs=[pl.BlockSpec((BN, F), lambda i: (i, 0)),
                  pl.BlockSpec((BN, F), lambda i: (i, 0)),
                  pl.BlockSpec((BN, F), lambda i: (i, 0)),
                  blk(F, 4 * LSTM_H), blk(LSTM_H, 4 * LSTM_H), blk(1, 4 * LSTM_H),
                  blk(F, 4 * LSTM_H), blk(LSTM_H, 4 * LSTM_H), blk(1, 4 * LSTM_H),
                  blk(1, 2 * LSTM_H), blk(1, 1),
                  blk(F, F), blk(1, F), blk(1, F), blk(1, 1),
                  pl.BlockSpec((1, BN), lambda i: (0, i)), blk(1, 1)],
        out_specs=[pl.BlockSpec((1, 1), lambda i: (0, 0))],
        out_shape=[jax.ShapeDtypeStruct((1, 1), _f32)],
    )(xs1, xs2, xs3, wif, whf, bf, wib, whb, bb,
      jkw, jkb, lin, linb, fc1, fc1b, fc2, fc2b)[0]


def kernel(x, edge_index, params):
    p = params
    loop = jnp.arange(N, dtype=jnp.int32)
    pad = jnp.full((EP - E2,), N, jnp.int32)
    src = jnp.concatenate([edge_index[0].astype(jnp.int32), loop, pad])
    dst = jnp.concatenate([edge_index[1].astype(jnp.int32), loop, pad])

    x_pad = jnp.zeros((N2, F), _f32).at[:N].set(x)
    zf = jnp.zeros((N2, F), _f32)
    zd = jnp.zeros((N2, 16), _f32)

    kron_a = jnp.kron(jnp.eye(H, dtype=_f32), jnp.ones((F // H, 1), _f32))
    kron_b = jnp.kron(jnp.eye(H, dtype=_f32), jnp.ones((1, F // H), _f32))
    B16 = jnp.zeros((16, F), _f32).at[:H].set(kron_b)

    h = x_pad
    xs = []
    for l in range(NL):
        A16 = jnp.zeros((F, 16), _f32).at[:, :H].set(
            kron_a * p['att%d' % l].reshape(F, 1))
        xl, xr = _mm2(h, p['Wl%d' % l], p['bl%d' % l].reshape(1, F),
                      p['Wr%d' % l], p['br%d' % l].reshape(1, F))
        gl, gr = _edge_gather(xl, xr, src, dst)
        al, gmax = _alpha_call(gl, gr, A16)
        w, pp = _wp_call(gl, al, gmax, B16)
        accs = _edge_scatter_f(w, dst, zf)
        accds = _edge_scatter_d(pp, dst, zd)
        h = _fin_call(accs, accds, B16, p['bias%d' % l].reshape(1, F))
        xs.append(h)

    bf = (p['bih_fwd'] + p['bhh_fwd']).reshape(1, 4 * LSTM_H)
    bb = (p['bih_bwd'] + p['bhh_bwd']).reshape(1, 4 * LSTM_H)
    fc2_pad = jnp.zeros((1, N2), _f32).at[:, :N].set(p['fc2_w'])

    out = _tail_call(
        xs[0], xs[1], xs[2],
        p['Wih_fwd'].T, p['Whh_fwd'].T, bf,
        p['Wih_bwd'].T, p['Whh_bwd'].T, bb,
        p['jk_att_w'], p['jk_att_b'].reshape(1, 1),
        p['lin_w'].T, p['lin_b'].reshape(1, F),
        p['fc1_w'], p['fc1_b'].reshape(1, 1),
        fc2_pad, p['fc2_b'].reshape(1, 1))
    return out.reshape(1)
